# baseline (speedup 1.0000x reference)
import functools

import numpy as np
import jax
import jax.numpy as jnp
from jax import lax
from jax.experimental import pallas as pl
from jax.experimental.pallas import tpu as pltpu
from jax.experimental.pallas import tpu_sc as plsc

_LANES = 16


def _pos_encoding(length, d_model, n=10000):
    d2 = d_model / 2
    position = np.arange(length)[:, np.newaxis]
    index = np.arange(int(d2))[np.newaxis, :]
    angle = position * np.power(n, -index / d2)
    return np.concatenate([np.sin(angle), np.cos(angle)], axis=-1).astype(np.float32)


def kernel(inputs, table):
    B, T = inputs.shape
    V, D = table.shape
    NW = 32
    CH = T // 2
    G = T // CH
    seqs_per_w = B // NW
    n_lane = D // _LANES
    scale = float(np.sqrt(D))

    pe = jnp.asarray(_pos_encoding(T, D))
    idx = inputs.reshape(NW, seqs_per_w, G, CH).astype(jnp.int32)

    mesh = plsc.VectorSubcoreMesh(core_axis_name="c", subcore_axis_name="s")

    @functools.partial(
        pl.kernel,
        mesh=mesh,
        out_type=jax.ShapeDtypeStruct((B, T, D), jnp.float32),
        compiler_params=pltpu.CompilerParams(use_tc_tiling_on_sc=False),
        scratch_types=[
            pltpu.VMEM((seqs_per_w, G, CH), jnp.int32),
            pltpu.VMEM((T, D), jnp.float32),
            pltpu.VMEM((T, D), jnp.float32),
            pltpu.SemaphoreType.DMA,
        ],
    )
    def emb(idx_hbm, table_hbm, pe_hbm, out_hbm, idx_v, pe_v, rows_v, sem):
        cid = lax.axis_index("c")
        sid = lax.axis_index("s")
        wid = sid * 2 + cid
        base = wid * seqs_per_w
        pltpu.sync_copy(idx_hbm.at[wid], idx_v)
        pltpu.sync_copy(pe_hbm, pe_v)

        def seq_body(s, carry):
            copies = [
                pltpu.async_copy(
                    table_hbm.at[idx_v.at[s, g]],
                    rows_v.at[pl.ds(g * CH, CH)],
                    sem,
                )
                for g in range(G)
            ]
            for c in copies:
                c.wait()

            def row_body(t, c2):
                for l in range(n_lane):
                    sl = pl.ds(l * _LANES, _LANES)
                    rows_v[t, sl] = rows_v[t, sl] * scale + pe_v[t, sl]
                return c2

            lax.fori_loop(0, T, row_body, 0)
            pltpu.sync_copy(rows_v, out_hbm.at[base + s])
            return carry

        lax.fori_loop(0, seqs_per_w, seq_body, 0)

    return emb(idx, table, pe)
